# baseline (speedup 1.0000x reference)
import jax
import jax.numpy as jnp
from jax.experimental import pallas as pl
from jax.experimental.pallas import tpu as pltpu

TOP_K = 2
NUM_EXPERTS = 64
HIDDEN = 2048
TOKENS = 8192

TILE_T = 512
N_TILES = TOKENS // TILE_T
NBUF = 4


def _top2_scores(logits):
    e_iota = jax.lax.broadcasted_iota(jnp.int32, logits.shape, 1).astype(jnp.float32)
    big = jnp.float32(NUM_EXPERTS)

    m1 = jnp.max(logits, axis=1, keepdims=True)
    i1 = jnp.min(jnp.where(logits == m1, e_iota, big), axis=1, keepdims=True)

    masked = jnp.where(e_iota == i1, -jnp.inf, logits)
    m2 = jnp.max(masked, axis=1, keepdims=True)
    i2 = jnp.min(jnp.where(masked == m2, e_iota, big), axis=1, keepdims=True)

    d = jnp.exp(m2 - m1)
    denom = 1.0 + d
    p1 = 1.0 / denom
    p2 = d / denom

    scores = jnp.where(e_iota == i1, p1, jnp.where(e_iota == i2, p2, 0.0))
    idx = jnp.concatenate([i1, i2], axis=1).astype(jnp.int32)
    return scores, idx


def _router_kernel(x_hbm, wt_ref, b_ref, scores_hbm, idx_hbm,
                   x_bufs, s_bufs, i_bufs, in_sems, s_sems, i_sems):
    wt = wt_ref[...]
    bias = b_ref[...]

    def in_copy(t, slot):
        return pltpu.make_async_copy(
            x_hbm.at[pl.ds(t * TILE_T, TILE_T), :], x_bufs.at[slot], in_sems.at[slot])

    def s_copy(t, slot):
        return pltpu.make_async_copy(
            s_bufs.at[slot], scores_hbm.at[pl.ds(t * TILE_T, TILE_T), :], s_sems.at[slot])

    def i_copy(t, slot):
        return pltpu.make_async_copy(
            i_bufs.at[slot], idx_hbm.at[pl.ds(t * TILE_T, TILE_T), :], i_sems.at[slot])

    for t in range(NBUF):
        in_copy(t, t).start()

    def body(t):
        slot = t % NBUF
        oslot = t % 2
        in_copy(t, slot).wait()
        logits = jnp.dot(x_bufs[slot], wt, preferred_element_type=jnp.float32) + bias
        scores, idx = _top2_scores(logits)

        if t >= 2:
            s_copy(t - 2, oslot).wait()
            i_copy(t - 2, oslot).wait()
        s_bufs[oslot] = scores
        i_bufs[oslot] = idx
        s_copy(t, oslot).start()
        i_copy(t, oslot).start()

        if t + NBUF < N_TILES:
            in_copy(t + NBUF, slot).start()

    for t in range(N_TILES):
        body(t)

    for t in (N_TILES - 2, N_TILES - 1):
        s_copy(t, t % 2).wait()
        i_copy(t, t % 2).wait()


@jax.jit
def kernel(hidden_states, W, b):
    x = hidden_states.reshape(-1, HIDDEN)
    wt = W.T
    b2 = b.reshape(1, NUM_EXPERTS)
    scores, idx = pl.pallas_call(
        _router_kernel,
        in_specs=[
            pl.BlockSpec(memory_space=pltpu.MemorySpace.HBM),
            pl.BlockSpec((HIDDEN, NUM_EXPERTS), lambda: (0, 0)),
            pl.BlockSpec((1, NUM_EXPERTS), lambda: (0, 0)),
        ],
        out_specs=[
            pl.BlockSpec(memory_space=pltpu.MemorySpace.HBM),
            pl.BlockSpec(memory_space=pltpu.MemorySpace.HBM),
        ],
        out_shape=[
            jax.ShapeDtypeStruct((TOKENS, NUM_EXPERTS), jnp.float32),
            jax.ShapeDtypeStruct((TOKENS, TOP_K), jnp.int32),
        ],
        scratch_shapes=[
            pltpu.VMEM((NBUF, TILE_T, HIDDEN), jnp.float32),
            pltpu.VMEM((2, TILE_T, NUM_EXPERTS), jnp.float32),
            pltpu.VMEM((2, TILE_T, TOP_K), jnp.int32),
            pltpu.SemaphoreType.DMA((NBUF,)),
            pltpu.SemaphoreType.DMA((2,)),
            pltpu.SemaphoreType.DMA((2,)),
        ],
    )(x, wt, b2)
    return scores, idx


# speedup vs baseline: 1.1129x; 1.1129x over previous
import jax
import jax.numpy as jnp
from jax.experimental import pallas as pl
from jax.experimental.pallas import tpu as pltpu

TOP_K = 2
NUM_EXPERTS = 64
HIDDEN = 2048
TOKENS = 8192

TILE_T = 512
N_TILES = TOKENS // TILE_T
NBUF = 4


def _top2_scores(logits):
    e_iota = jax.lax.broadcasted_iota(jnp.int32, logits.shape, 1).astype(jnp.float32)
    big = jnp.float32(NUM_EXPERTS)

    m1 = jnp.max(logits, axis=1, keepdims=True)
    i1 = jnp.min(jnp.where(logits == m1, e_iota, big), axis=1, keepdims=True)

    masked = jnp.where(e_iota == i1, -jnp.inf, logits)
    m2 = jnp.max(masked, axis=1, keepdims=True)
    i2 = jnp.min(jnp.where(masked == m2, e_iota, big), axis=1, keepdims=True)

    d = jnp.exp(m2 - m1)
    denom = 1.0 + d
    p1 = 1.0 / denom
    p2 = d / denom

    scores = jnp.where(e_iota == i1, p1, jnp.where(e_iota == i2, p2, 0.0))
    idx = jnp.concatenate([i1, i2], axis=1).astype(jnp.int32)
    return scores, idx


def _router_kernel(x_hbm, wt_ref, b_ref, scores_hbm, idx_hbm,
                   x_bufs, s_bufs, i_bufs, in_sems, s_sems, i_sems):
    wt = wt_ref[...]
    bias = b_ref[...]

    def in_copy(t, slot):
        return pltpu.make_async_copy(
            x_hbm.at[pl.ds(t * TILE_T, TILE_T), :], x_bufs.at[slot], in_sems.at[slot])

    def s_copy(t, slot):
        return pltpu.make_async_copy(
            s_bufs.at[slot], scores_hbm.at[pl.ds(t * TILE_T, TILE_T), :], s_sems.at[slot])

    def i_copy(t, slot):
        return pltpu.make_async_copy(
            i_bufs.at[slot], idx_hbm.at[pl.ds(t * TILE_T, TILE_T), :], i_sems.at[slot])

    for t in range(NBUF):
        in_copy(t, t).start()

    def body(t, _):
        slot = jax.lax.rem(t, NBUF)
        oslot = jax.lax.rem(t, 2)
        in_copy(t, slot).wait()
        logits = jnp.dot(x_bufs[slot], wt, preferred_element_type=jnp.float32) + bias
        scores, idx = _top2_scores(logits)

        @pl.when(t >= 2)
        def _():
            s_copy(t - 2, oslot).wait()
            i_copy(t - 2, oslot).wait()
        s_bufs[oslot] = scores
        i_bufs[oslot] = idx
        s_copy(t, oslot).start()
        i_copy(t, oslot).start()

        @pl.when(t + NBUF < N_TILES)
        def _():
            in_copy(t + NBUF, slot).start()
        return _

    jax.lax.fori_loop(0, N_TILES, body, None)

    for t in (N_TILES - 2, N_TILES - 1):
        s_copy(t, t % 2).wait()
        i_copy(t, t % 2).wait()


@jax.jit
def kernel(hidden_states, W, b):
    x = hidden_states.reshape(-1, HIDDEN)
    wt = W.T
    b2 = b.reshape(1, NUM_EXPERTS)
    scores, idx = pl.pallas_call(
        _router_kernel,
        in_specs=[
            pl.BlockSpec(memory_space=pltpu.MemorySpace.HBM),
            pl.BlockSpec((HIDDEN, NUM_EXPERTS), lambda: (0, 0)),
            pl.BlockSpec((1, NUM_EXPERTS), lambda: (0, 0)),
        ],
        out_specs=[
            pl.BlockSpec(memory_space=pltpu.MemorySpace.HBM),
            pl.BlockSpec(memory_space=pltpu.MemorySpace.HBM),
        ],
        out_shape=[
            jax.ShapeDtypeStruct((TOKENS, NUM_EXPERTS), jnp.float32),
            jax.ShapeDtypeStruct((TOKENS, TOP_K), jnp.int32),
        ],
        scratch_shapes=[
            pltpu.VMEM((NBUF, TILE_T, HIDDEN), jnp.float32),
            pltpu.VMEM((2, TILE_T, NUM_EXPERTS), jnp.float32),
            pltpu.VMEM((2, TILE_T, TOP_K), jnp.int32),
            pltpu.SemaphoreType.DMA((NBUF,)),
            pltpu.SemaphoreType.DMA((2,)),
            pltpu.SemaphoreType.DMA((2,)),
        ],
    )(x, wt, b2)
    return scores, idx


# speedup vs baseline: 1.1258x; 1.0116x over previous
import jax
import jax.numpy as jnp
from jax.experimental import pallas as pl
from jax.experimental.pallas import tpu as pltpu

TOP_K = 2
NUM_EXPERTS = 64
HIDDEN = 2048
TOKENS = 8192

TILE_T = 512
N_TILES = TOKENS // TILE_T
NBUF = 3


def _top2_scores(logits):
    e_iota = jax.lax.broadcasted_iota(jnp.int32, logits.shape, 1).astype(jnp.float32)
    big = jnp.float32(NUM_EXPERTS)

    m1 = jnp.max(logits, axis=1, keepdims=True)
    i1 = jnp.min(jnp.where(logits == m1, e_iota, big), axis=1, keepdims=True)

    masked = jnp.where(e_iota == i1, -jnp.inf, logits)
    m2 = jnp.max(masked, axis=1, keepdims=True)
    i2 = jnp.min(jnp.where(masked == m2, e_iota, big), axis=1, keepdims=True)

    d = jnp.exp(m2 - m1)
    denom = 1.0 + d
    p1 = 1.0 / denom
    p2 = d / denom

    scores = jnp.where(e_iota == i1, p1, jnp.where(e_iota == i2, p2, 0.0))
    idx = jnp.concatenate([i1, i2], axis=1).astype(jnp.int32)
    return scores, idx


def _router_kernel(x_hbm, wt_ref, b_ref, scores_hbm, idx_hbm,
                   x_bufs, s_bufs, i_bufs, in_sems, s_sems, i_sems):
    wt = wt_ref[...]
    bias = b_ref[...]

    def in_copy(t, slot):
        return pltpu.make_async_copy(
            x_hbm.at[pl.ds(t * TILE_T, TILE_T), :], x_bufs.at[slot], in_sems.at[slot])

    def s_copy(t, slot):
        return pltpu.make_async_copy(
            s_bufs.at[slot], scores_hbm.at[pl.ds(t * TILE_T, TILE_T), :], s_sems.at[slot])

    def i_copy(t, slot):
        return pltpu.make_async_copy(
            i_bufs.at[slot], idx_hbm.at[pl.ds(t * TILE_T, TILE_T), :], i_sems.at[slot])

    for t in range(NBUF):
        in_copy(t, t).start()

    def body(t, _):
        slot = jax.lax.rem(t, NBUF)
        oslot = jax.lax.rem(t, 2)
        in_copy(t, slot).wait()
        logits = jnp.dot(x_bufs[slot], wt, preferred_element_type=jnp.float32) + bias
        scores, idx = _top2_scores(logits)

        @pl.when(t >= 2)
        def _():
            s_copy(t - 2, oslot).wait()
            i_copy(t - 2, oslot).wait()
        s_bufs[oslot] = scores
        i_bufs[oslot] = idx
        s_copy(t, oslot).start()
        i_copy(t, oslot).start()

        @pl.when(t + NBUF < N_TILES)
        def _():
            in_copy(t + NBUF, slot).start()
        return _

    jax.lax.fori_loop(0, N_TILES, body, None)

    for t in (N_TILES - 2, N_TILES - 1):
        s_copy(t, t % 2).wait()
        i_copy(t, t % 2).wait()


@jax.jit
def kernel(hidden_states, W, b):
    x = hidden_states.reshape(-1, HIDDEN)
    wt = W.T
    b2 = b.reshape(1, NUM_EXPERTS)
    scores, idx = pl.pallas_call(
        _router_kernel,
        in_specs=[
            pl.BlockSpec(memory_space=pltpu.MemorySpace.HBM),
            pl.BlockSpec((HIDDEN, NUM_EXPERTS), lambda: (0, 0)),
            pl.BlockSpec((1, NUM_EXPERTS), lambda: (0, 0)),
        ],
        out_specs=[
            pl.BlockSpec(memory_space=pltpu.MemorySpace.HBM),
            pl.BlockSpec(memory_space=pltpu.MemorySpace.HBM),
        ],
        out_shape=[
            jax.ShapeDtypeStruct((TOKENS, NUM_EXPERTS), jnp.float32),
            jax.ShapeDtypeStruct((TOKENS, TOP_K), jnp.int32),
        ],
        scratch_shapes=[
            pltpu.VMEM((NBUF, TILE_T, HIDDEN), jnp.float32),
            pltpu.VMEM((2, TILE_T, NUM_EXPERTS), jnp.float32),
            pltpu.VMEM((2, TILE_T, TOP_K), jnp.int32),
            pltpu.SemaphoreType.DMA((NBUF,)),
            pltpu.SemaphoreType.DMA((2,)),
            pltpu.SemaphoreType.DMA((2,)),
        ],
    )(x, wt, b2)
    return scores, idx
